# baseline (speedup 1.0000x reference)
import functools

import jax
import jax.numpy as jnp
from jax import lax
from jax.experimental import pallas as pl
from jax.experimental.pallas import tpu as pltpu
from jax.experimental.pallas import tpu_sc as plsc

DIM = 100000
RANK = 64
BATCH = 16384
NC = 2
NS = 16
E = 8
EB = BATCH // E
RPW = RANK // NC // NS
TILE_B = BATCH // NS
UNROLL = 8
NSTAGE = NS * RPW
RING = 8


def _gather_pass(idx_hbm, rowbuf, idxq, gu, sem, row_cp, mul):
    nxt = pltpu.async_copy(idx_hbm.at[pl.ds(0, EB)], idxq.at[0], sem)
    for e in range(E):
        cur = nxt
        if e + 1 < E:
            nxt = pltpu.async_copy(
                idx_hbm.at[pl.ds((e + 1) * EB, EB)],
                idxq.at[(e + 1) % 2], sem)
        cur.wait()
        if e == 0:
            row_cp.wait()
        sl = e % 2
        if mul:
            @plsc.parallel_loop(0, EB, step=16, unroll=UNROLL)
            def vbody(o, e=e, sl=sl):
                iv = idxq[sl, pl.ds(o, 16)]
                g = plsc.load_gather(rowbuf, [iv])
                gu[pl.ds(e * EB + o, 16)] = g * gu[pl.ds(e * EB + o, 16)]
        else:
            @plsc.parallel_loop(0, EB, step=16, unroll=UNROLL)
            def ubody(o, e=e, sl=sl):
                iv = idxq[sl, pl.ds(o, 16)]
                gu[pl.ds(e * EB + o, 16)] = plsc.load_gather(rowbuf, [iv])


def _sc_body(i1_hbm, i2_hbm, u_hbm, v_hbm, bu_hbm, bv_hbm,
             out_hbm, stage_hbm, rowbuf, idxq, gu, tmp, outbuf, sem, sem2):
    scid = lax.axis_index("c")
    sid = lax.axis_index("s")
    gbase = sid * TILE_B

    stage_cp = None
    for rloc in range(RPW):
        r = scid * (RANK // NC) + sid * RPW + rloc

        row_cp = pltpu.async_copy(u_hbm.at[r], rowbuf, sem2)
        if stage_cp is not None:
            stage_cp.wait()
        _gather_pass(i1_hbm, rowbuf, idxq, gu, sem, row_cp, mul=False)

        row_cp = pltpu.async_copy(v_hbm.at[r], rowbuf, sem2)
        _gather_pass(i2_hbm, rowbuf, idxq, gu, sem, row_cp, mul=True)

        stage_cp = pltpu.async_copy(
            gu, stage_hbm.at[scid * NSTAGE + sid * RPW + rloc], sem2)

    stage_cp.wait()
    plsc.subcore_barrier()

    def _back(bias_hbm, idx_hbm, sbase, obase):
        bias_cp = pltpu.async_copy(bias_hbm, rowbuf, sem2)
        bidx_cp = pltpu.async_copy(
            idx_hbm.at[pl.ds(gbase, TILE_B)],
            idxq.at[0, pl.ds(0, TILE_B)], sem)

        copies = [pltpu.async_copy(
            stage_hbm.at[sbase + j, pl.ds(gbase, TILE_B)],
            tmp.at[pl.ds((j % RING) * TILE_B, TILE_B)], sem)
            for j in range(RING)]
        for j in range(NSTAGE):
            copies[j].wait()

            if j == 0:
                @plsc.parallel_loop(0, TILE_B, step=16, unroll=UNROLL)
                def red0(o, j=j):
                    outbuf[pl.ds(o, 16)] = tmp[
                        pl.ds((j % RING) * TILE_B + o, 16)]
            else:
                @plsc.parallel_loop(0, TILE_B, step=16, unroll=UNROLL)
                def redbody(o, j=j):
                    outbuf[pl.ds(o, 16)] = (
                        outbuf[pl.ds(o, 16)]
                        + tmp[pl.ds((j % RING) * TILE_B + o, 16)])
            if j + RING < NSTAGE:
                copies.append(pltpu.async_copy(
                    stage_hbm.at[sbase + j + RING, pl.ds(gbase, TILE_B)],
                    tmp.at[pl.ds((j % RING) * TILE_B, TILE_B)], sem))

        bias_cp.wait()
        bidx_cp.wait()

        @plsc.parallel_loop(0, TILE_B, step=16, unroll=UNROLL)
        def abody(o):
            iv = idxq[0, pl.ds(o, 16)]
            outbuf[pl.ds(o, 16)] = (
                outbuf[pl.ds(o, 16)] + plsc.load_gather(rowbuf, [iv]))

        pltpu.sync_copy(outbuf, out_hbm.at[pl.ds(obase + gbase, TILE_B)])

    @pl.when(scid == 0)
    def _back_u():
        _back(bu_hbm, i1_hbm, 0, 0)

    @pl.when(scid == 1)
    def _back_v():
        _back(bv_hbm, i2_hbm, NSTAGE, BATCH)


@functools.partial(
    pl.kernel,
    out_type=(
        jax.ShapeDtypeStruct((NC * BATCH,), jnp.float32),
        jax.ShapeDtypeStruct((NC * NSTAGE, BATCH), jnp.float32),
    ),
    mesh=plsc.VectorSubcoreMesh(core_axis_name="c", subcore_axis_name="s"),
    compiler_params=pltpu.CompilerParams(
        needs_layout_passes=False, use_tc_tiling_on_sc=True),
    scratch_types=[
        pltpu.VMEM((DIM,), jnp.float32),
        pltpu.VMEM((2, EB), jnp.int32),
        pltpu.VMEM((BATCH,), jnp.float32),
        pltpu.VMEM((RING * TILE_B,), jnp.float32),
        pltpu.VMEM((TILE_B,), jnp.float32),
        pltpu.SemaphoreType.DMA,
        pltpu.SemaphoreType.DMA,
    ],
)
def _sc_kernel(i1_hbm, i2_hbm, u_hbm, v_hbm, bu_hbm, bv_hbm,
               out_hbm, stage_hbm, *scratch):
    _sc_body(i1_hbm, i2_hbm, u_hbm, v_hbm, bu_hbm, bv_hbm,
             out_hbm, stage_hbm, *scratch)


def kernel(x, U_w, V_w, bias_U, bias_V):
    i1 = x[:, 0].astype(jnp.int32)
    i2 = x[:, 1].astype(jnp.int32)
    part, _ = _sc_kernel(i1, i2, U_w, V_w, bias_U, bias_V)
    part = part.reshape(NC, BATCH)
    return (part[0] + part[1]).reshape(BATCH, 1)


# speedup vs baseline: 1.0211x; 1.0211x over previous
import functools

import jax
import jax.numpy as jnp
from jax import lax
from jax.experimental import pallas as pl
from jax.experimental.pallas import tpu as pltpu
from jax.experimental.pallas import tpu_sc as plsc

DIM = 100000
RANK = 64
BATCH = 16384
NC = 2
NS = 16
E = 8
EB = BATCH // E
RPW = RANK // NC // NS
TILE_B = BATCH // NS
UNROLL = 8
NSTAGE = NS * RPW
RG = 4


def _gather_pass(idx_hbm, rowbuf, idxq, gu, semi, row_cp, mul):
    nxt = pltpu.async_copy(idx_hbm.at[pl.ds(0, EB)], idxq.at[0],
                           semi.at[0])
    for e in range(E):
        cur = nxt
        if e + 1 < E:
            nxt = pltpu.async_copy(
                idx_hbm.at[pl.ds((e + 1) * EB, EB)],
                idxq.at[(e + 1) % 2], semi.at[(e + 1) % 2])
        cur.wait()
        if e == 0:
            row_cp.wait()
        sl = e % 2
        if mul:
            @plsc.parallel_loop(0, EB, step=16, unroll=UNROLL)
            def vbody(o, e=e, sl=sl):
                iv = idxq[sl, pl.ds(o, 16)]
                g = plsc.load_gather(rowbuf, [iv])
                gu[pl.ds(e * EB + o, 16)] = g * gu[pl.ds(e * EB + o, 16)]
        else:
            @plsc.parallel_loop(0, EB, step=16, unroll=UNROLL)
            def ubody(o, e=e, sl=sl):
                iv = idxq[sl, pl.ds(o, 16)]
                gu[pl.ds(e * EB + o, 16)] = plsc.load_gather(rowbuf, [iv])


def _sc_body(i1_hbm, i2_hbm, u_hbm, v_hbm, bu_hbm, bv_hbm,
             out_hbm, stage_hbm, rowbuf, idxq, gu, tmp, outbuf,
             semi, semt, semr, sems, semb):
    scid = lax.axis_index("c")
    sid = lax.axis_index("s")
    gbase = sid * TILE_B

    stage_cp = None
    for rloc in range(RPW):
        r = scid * (RANK // NC) + sid * RPW + rloc

        row_cp = pltpu.async_copy(u_hbm.at[r], rowbuf, semr)
        if stage_cp is not None:
            stage_cp.wait()
        _gather_pass(i1_hbm, rowbuf, idxq, gu, semi, row_cp, mul=False)

        row_cp = pltpu.async_copy(v_hbm.at[r], rowbuf, semr)
        _gather_pass(i2_hbm, rowbuf, idxq, gu, semi, row_cp, mul=True)

        stage_cp = pltpu.async_copy(
            gu, stage_hbm.at[scid * NSTAGE + sid * RPW + rloc], sems)

    stage_cp.wait()
    plsc.subcore_barrier()

    def _back(bias_hbm, idx_hbm, sbase, obase):
        bias_cp = pltpu.async_copy(bias_hbm, rowbuf, semb)
        bidx_cp = pltpu.async_copy(
            idx_hbm.at[pl.ds(gbase, TILE_B)],
            idxq.at[0, pl.ds(0, TILE_B)], semi.at[0])

        ngroups = NSTAGE // RG
        copies = [pltpu.async_copy(
            stage_hbm.at[pl.ds(sbase + g * RG, RG), pl.ds(gbase, TILE_B)],
            tmp.at[g % 2], semt.at[g % 2]) for g in range(2)]
        for g in range(ngroups):
            copies[g].wait()
            sl = g % 2

            if g == 0:
                @plsc.parallel_loop(0, TILE_B, step=16, unroll=UNROLL)
                def red0(o, sl=sl):
                    outbuf[pl.ds(o, 16)] = (
                        (tmp[sl, 0, pl.ds(o, 16)]
                         + tmp[sl, 1, pl.ds(o, 16)])
                        + (tmp[sl, 2, pl.ds(o, 16)]
                           + tmp[sl, 3, pl.ds(o, 16)]))
            else:
                @plsc.parallel_loop(0, TILE_B, step=16, unroll=UNROLL)
                def redbody(o, sl=sl):
                    outbuf[pl.ds(o, 16)] = outbuf[pl.ds(o, 16)] + (
                        (tmp[sl, 0, pl.ds(o, 16)]
                         + tmp[sl, 1, pl.ds(o, 16)])
                        + (tmp[sl, 2, pl.ds(o, 16)]
                           + tmp[sl, 3, pl.ds(o, 16)]))
            if g + 2 < ngroups:
                copies.append(pltpu.async_copy(
                    stage_hbm.at[pl.ds(sbase + (g + 2) * RG, RG),
                                 pl.ds(gbase, TILE_B)],
                    tmp.at[g % 2], semt.at[g % 2]))

        bias_cp.wait()
        bidx_cp.wait()

        @plsc.parallel_loop(0, TILE_B, step=16, unroll=UNROLL)
        def abody(o):
            iv = idxq[0, pl.ds(o, 16)]
            outbuf[pl.ds(o, 16)] = (
                outbuf[pl.ds(o, 16)] + plsc.load_gather(rowbuf, [iv]))

        pltpu.sync_copy(outbuf, out_hbm.at[pl.ds(obase + gbase, TILE_B)])

    @pl.when(scid == 0)
    def _back_u():
        _back(bu_hbm, i1_hbm, 0, 0)

    @pl.when(scid == 1)
    def _back_v():
        _back(bv_hbm, i2_hbm, NSTAGE, BATCH)


@functools.partial(
    pl.kernel,
    out_type=(
        jax.ShapeDtypeStruct((NC * BATCH,), jnp.float32),
        jax.ShapeDtypeStruct((NC * NSTAGE, BATCH), jnp.float32),
    ),
    mesh=plsc.VectorSubcoreMesh(core_axis_name="c", subcore_axis_name="s"),
    compiler_params=pltpu.CompilerParams(
        needs_layout_passes=False, use_tc_tiling_on_sc=True),
    scratch_types=[
        pltpu.VMEM((DIM,), jnp.float32),
        pltpu.VMEM((2, EB), jnp.int32),
        pltpu.VMEM((BATCH,), jnp.float32),
        pltpu.VMEM((2, RG, TILE_B), jnp.float32),
        pltpu.VMEM((TILE_B,), jnp.float32),
        pltpu.SemaphoreType.DMA((2,)),
        pltpu.SemaphoreType.DMA((2,)),
        pltpu.SemaphoreType.DMA,
        pltpu.SemaphoreType.DMA,
        pltpu.SemaphoreType.DMA,
    ],
)
def _sc_kernel(i1_hbm, i2_hbm, u_hbm, v_hbm, bu_hbm, bv_hbm,
               out_hbm, stage_hbm, *scratch):
    _sc_body(i1_hbm, i2_hbm, u_hbm, v_hbm, bu_hbm, bv_hbm,
             out_hbm, stage_hbm, *scratch)


def kernel(x, U_w, V_w, bias_U, bias_V):
    i1 = x[:, 0].astype(jnp.int32)
    i2 = x[:, 1].astype(jnp.int32)
    part, _ = _sc_kernel(i1, i2, U_w, V_w, bias_U, bias_V)
    part = part.reshape(NC, BATCH)
    return (part[0] + part[1]).reshape(BATCH, 1)


# speedup vs baseline: 1.0605x; 1.0386x over previous
import functools

import jax
import jax.numpy as jnp
from jax import lax
from jax.experimental import pallas as pl
from jax.experimental.pallas import tpu as pltpu
from jax.experimental.pallas import tpu_sc as plsc

DIM = 100000
RANK = 64
BATCH = 16384
NC = 2
NS = 16
E = 4
EB = BATCH // E
RPW = RANK // NC // NS
TILE_B = BATCH // NS
UNROLL = 8
NSTAGE = NS * RPW
RG = 2


def _gather_pass(idx_hbm, rowbuf, idxq, gu, semi, row_cp, mul):
    nxt = pltpu.async_copy(idx_hbm.at[pl.ds(0, EB)], idxq.at[0],
                           semi.at[0])
    for e in range(E):
        cur = nxt
        if e + 1 < E:
            nxt = pltpu.async_copy(
                idx_hbm.at[pl.ds((e + 1) * EB, EB)],
                idxq.at[(e + 1) % 2], semi.at[(e + 1) % 2])
        cur.wait()
        if e == 0:
            row_cp.wait()
        sl = e % 2
        if mul:
            @plsc.parallel_loop(0, EB, step=16, unroll=UNROLL)
            def vbody(o, e=e, sl=sl):
                iv = idxq[sl, pl.ds(o, 16)]
                g = plsc.load_gather(rowbuf, [iv])
                t = e * EB + o
                gu[t // 128, pl.ds(t % 128, 16)] = (
                    g * gu[t // 128, pl.ds(t % 128, 16)])
        else:
            @plsc.parallel_loop(0, EB, step=16, unroll=UNROLL)
            def ubody(o, e=e, sl=sl):
                iv = idxq[sl, pl.ds(o, 16)]
                t = e * EB + o
                gu[t // 128, pl.ds(t % 128, 16)] = plsc.load_gather(
                    rowbuf, [iv])


def _sc_body(i1_hbm, i2_hbm, u_hbm, v_hbm, bu_hbm, bv_hbm,
             out_hbm, stage_hbm, rowbuf, idxq, gu, tmp, outbuf,
             semi, semt, semr, sems, semb):
    scid = lax.axis_index("c")
    sid = lax.axis_index("s")
    gbase = sid * TILE_B

    stage_cp = None
    for rloc in range(RPW):
        r = scid * (RANK // NC) + sid * RPW + rloc

        row_cp = pltpu.async_copy(u_hbm.at[r], rowbuf, semr)
        if stage_cp is not None:
            stage_cp.wait()
        _gather_pass(i1_hbm, rowbuf, idxq, gu, semi, row_cp, mul=False)

        row_cp = pltpu.async_copy(v_hbm.at[r], rowbuf, semr)
        _gather_pass(i2_hbm, rowbuf, idxq, gu, semi, row_cp, mul=True)

        stage_cp = pltpu.async_copy(
            gu, stage_hbm.at[scid * NSTAGE + sid * RPW + rloc], sems)

    stage_cp.wait()
    plsc.subcore_barrier()

    def _back(bias_hbm, idx_hbm, sbase, obase):
        bias_cp = pltpu.async_copy(bias_hbm, rowbuf, semb)
        bidx_cp = pltpu.async_copy(
            idx_hbm.at[pl.ds(gbase, TILE_B)],
            idxq.at[0, pl.ds(0, TILE_B)], semi.at[0])

        prow = sid * (TILE_B // 128)
        ngroups = NSTAGE // RG
        copies = [pltpu.async_copy(
            stage_hbm.at[pl.ds(sbase + g * RG, RG),
                         pl.ds(prow, TILE_B // 128)],
            tmp.at[g % 2], semt.at[g % 2]) for g in range(2)]
        for g in range(ngroups):
            copies[g].wait()
            sl = g % 2

            if g == 0:
                @plsc.parallel_loop(0, TILE_B, step=16, unroll=UNROLL)
                def red0(o, sl=sl):
                    outbuf[pl.ds(o, 16)] = (
                        tmp[sl, 0, o // 128, pl.ds(o % 128, 16)]
                        + tmp[sl, 1, o // 128, pl.ds(o % 128, 16)])
            else:
                @plsc.parallel_loop(0, TILE_B, step=16, unroll=UNROLL)
                def redbody(o, sl=sl):
                    outbuf[pl.ds(o, 16)] = outbuf[pl.ds(o, 16)] + (
                        tmp[sl, 0, o // 128, pl.ds(o % 128, 16)]
                        + tmp[sl, 1, o // 128, pl.ds(o % 128, 16)])
            if g + 2 < ngroups:
                copies.append(pltpu.async_copy(
                    stage_hbm.at[pl.ds(sbase + (g + 2) * RG, RG),
                                 pl.ds(prow, TILE_B // 128)],
                    tmp.at[g % 2], semt.at[g % 2]))

        bias_cp.wait()
        bidx_cp.wait()

        @plsc.parallel_loop(0, TILE_B, step=16, unroll=UNROLL)
        def abody(o):
            iv = idxq[0, pl.ds(o, 16)]
            outbuf[pl.ds(o, 16)] = (
                outbuf[pl.ds(o, 16)] + plsc.load_gather(rowbuf, [iv]))

        pltpu.sync_copy(outbuf, out_hbm.at[pl.ds(obase + gbase, TILE_B)])

    @pl.when(scid == 0)
    def _back_u():
        _back(bu_hbm, i1_hbm, 0, 0)

    @pl.when(scid == 1)
    def _back_v():
        _back(bv_hbm, i2_hbm, NSTAGE, BATCH)


@functools.partial(
    pl.kernel,
    out_type=(
        jax.ShapeDtypeStruct((NC * BATCH,), jnp.float32),
        jax.ShapeDtypeStruct((NC * NSTAGE, 128, 128), jnp.float32),
    ),
    mesh=plsc.VectorSubcoreMesh(core_axis_name="c", subcore_axis_name="s"),
    compiler_params=pltpu.CompilerParams(
        needs_layout_passes=False, use_tc_tiling_on_sc=True),
    scratch_types=[
        pltpu.VMEM((DIM,), jnp.float32),
        pltpu.VMEM((2, EB), jnp.int32),
        pltpu.VMEM((128, 128), jnp.float32),
        pltpu.VMEM((2, RG, TILE_B // 128, 128), jnp.float32),
        pltpu.VMEM((TILE_B,), jnp.float32),
        pltpu.SemaphoreType.DMA((2,)),
        pltpu.SemaphoreType.DMA((2,)),
        pltpu.SemaphoreType.DMA,
        pltpu.SemaphoreType.DMA,
        pltpu.SemaphoreType.DMA,
    ],
)
def _sc_kernel(i1_hbm, i2_hbm, u_hbm, v_hbm, bu_hbm, bv_hbm,
               out_hbm, stage_hbm, *scratch):
    _sc_body(i1_hbm, i2_hbm, u_hbm, v_hbm, bu_hbm, bv_hbm,
             out_hbm, stage_hbm, *scratch)


def kernel(x, U_w, V_w, bias_U, bias_V):
    i1 = x[:, 0].astype(jnp.int32)
    i2 = x[:, 1].astype(jnp.int32)
    part, _ = _sc_kernel(i1, i2, U_w, V_w, bias_U, bias_V)
    part = part.reshape(NC, BATCH)
    return (part[0] + part[1]).reshape(BATCH, 1)


# speedup vs baseline: 1.1638x; 1.0975x over previous
import functools

import jax
import jax.numpy as jnp
from jax import lax
from jax.experimental import pallas as pl
from jax.experimental.pallas import tpu as pltpu
from jax.experimental.pallas import tpu_sc as plsc

DIM = 100000
RANK = 64
BATCH = 16384
NC = 2
NS = 16
E = 4
EB = BATCH // E
RPW = RANK // NC // NS
TILE_B = BATCH // NS
UNROLL = 8
NSTAGE = NS * RPW
RG = 2


def _gather_pass(idx_hbm, rowbuf, idxq, gu, semi, row_cp, mul):
    nxt = pltpu.async_copy(idx_hbm.at[pl.ds(0, EB)], idxq.at[pl.ds(0, EB)],
                           semi.at[0])
    for e in range(E):
        cur = nxt
        if e + 1 < E:
            nxt = pltpu.async_copy(
                idx_hbm.at[pl.ds((e + 1) * EB, EB)],
                idxq.at[pl.ds(((e + 1) % 2) * EB, EB)], semi.at[(e + 1) % 2])
        cur.wait()
        if e == 0:
            row_cp.wait()
        sl = e % 2
        if mul:
            @plsc.parallel_loop(0, EB, step=16, unroll=UNROLL)
            def vbody(o, e=e, sl=sl):
                iv = idxq[pl.ds(sl * EB + o, 16)]
                g = plsc.load_gather(rowbuf, [iv])
                t = e * EB + o
                gu[t // 128, pl.ds(t % 128, 16)] = (
                    g * gu[t // 128, pl.ds(t % 128, 16)])
        else:
            @plsc.parallel_loop(0, EB, step=16, unroll=UNROLL)
            def ubody(o, e=e, sl=sl):
                iv = idxq[pl.ds(sl * EB + o, 16)]
                t = e * EB + o
                gu[t // 128, pl.ds(t % 128, 16)] = plsc.load_gather(
                    rowbuf, [iv])


def _sc_body(i1_hbm, i2_hbm, u_hbm, v_hbm, bu_hbm, bv_hbm,
             out_hbm, rowbuf, idxq, gu, tmp, outbuf, outbuf2,
             semi, semt, semr, sems, semb, stage_hbm):
    scid = lax.axis_index("c")
    sid = lax.axis_index("s")
    gbase = sid * TILE_B

    stage_cp = None
    for rloc in range(RPW):
        r = scid * (RANK // NC) + sid * RPW + rloc

        row_cp = pltpu.async_copy(u_hbm.at[r], rowbuf, semr)
        if stage_cp is not None:
            stage_cp.wait()
        _gather_pass(i1_hbm, rowbuf, idxq, gu, semi, row_cp, mul=False)

        row_cp = pltpu.async_copy(v_hbm.at[r], rowbuf, semr)
        _gather_pass(i2_hbm, rowbuf, idxq, gu, semi, row_cp, mul=True)

        stage_cp = pltpu.async_copy(
            gu, stage_hbm.at[scid * NSTAGE + sid * RPW + rloc], sems)

    stage_cp.wait()
    plsc.subcore_barrier()

    def _back(bias_hbm, idx_hbm, sbase, obase):
        pltpu.sync_copy(idx_hbm.at[pl.ds(gbase, TILE_B)],
                        idxq.at[pl.ds(0, TILE_B)])
        bias_cp = pltpu.async_copy(
            bias_hbm.at[idxq.at[pl.ds(0, TILE_B)]],
            outbuf2, semb)

        prow = sid * (TILE_B // 128)
        ngroups = NSTAGE // RG
        copies = [pltpu.async_copy(
            stage_hbm.at[pl.ds(sbase + g * RG, RG),
                         pl.ds(prow, TILE_B // 128)],
            tmp.at[g % 2], semt.at[g % 2]) for g in range(2)]
        for g in range(ngroups):
            copies[g].wait()
            sl = g % 2

            if g == 0:
                @plsc.parallel_loop(0, TILE_B, step=16, unroll=UNROLL)
                def red0(o, sl=sl):
                    outbuf[pl.ds(o, 16)] = (
                        tmp[sl, 0, o // 128, pl.ds(o % 128, 16)]
                        + tmp[sl, 1, o // 128, pl.ds(o % 128, 16)])
            else:
                @plsc.parallel_loop(0, TILE_B, step=16, unroll=UNROLL)
                def redbody(o, sl=sl):
                    outbuf[pl.ds(o, 16)] = outbuf[pl.ds(o, 16)] + (
                        tmp[sl, 0, o // 128, pl.ds(o % 128, 16)]
                        + tmp[sl, 1, o // 128, pl.ds(o % 128, 16)])
            if g + 2 < ngroups:
                copies.append(pltpu.async_copy(
                    stage_hbm.at[pl.ds(sbase + (g + 2) * RG, RG),
                                 pl.ds(prow, TILE_B // 128)],
                    tmp.at[g % 2], semt.at[g % 2]))

        bias_cp.wait()

        @plsc.parallel_loop(0, TILE_B, step=16, unroll=UNROLL)
        def abody(o):
            outbuf[pl.ds(o, 16)] = (
                outbuf[pl.ds(o, 16)] + outbuf2[pl.ds(o, 16)])

        pltpu.sync_copy(outbuf, out_hbm.at[pl.ds(obase + gbase, TILE_B)])

    @pl.when(scid == 0)
    def _back_u():
        _back(bu_hbm, i1_hbm, 0, 0)

    @pl.when(scid == 1)
    def _back_v():
        _back(bv_hbm, i2_hbm, NSTAGE, BATCH)


@functools.partial(
    pl.kernel,
    out_type=jax.ShapeDtypeStruct((NC * BATCH,), jnp.float32),
    mesh=plsc.VectorSubcoreMesh(core_axis_name="c", subcore_axis_name="s"),
    compiler_params=pltpu.CompilerParams(
        needs_layout_passes=False, use_tc_tiling_on_sc=True),
    scratch_types=[
        pltpu.VMEM((DIM,), jnp.float32),
        pltpu.VMEM((2 * EB,), jnp.int32),
        pltpu.VMEM((128, 128), jnp.float32),
        pltpu.VMEM((2, RG, TILE_B // 128, 128), jnp.float32),
        pltpu.VMEM((TILE_B,), jnp.float32),
        pltpu.VMEM((TILE_B,), jnp.float32),
        pltpu.SemaphoreType.DMA((2,)),
        pltpu.SemaphoreType.DMA((2,)),
        pltpu.SemaphoreType.DMA,
        pltpu.SemaphoreType.DMA,
        pltpu.SemaphoreType.DMA,
        pltpu.HBM((NC * NSTAGE, 128, 128), jnp.float32),
    ],
)
def _sc_kernel(i1_hbm, i2_hbm, u_hbm, v_hbm, bu_hbm, bv_hbm,
               out_hbm, *scratch):
    _sc_body(i1_hbm, i2_hbm, u_hbm, v_hbm, bu_hbm, bv_hbm,
             out_hbm, *scratch)


def kernel(x, U_w, V_w, bias_U, bias_V):
    i1 = x[:, 0].astype(jnp.int32)
    i2 = x[:, 1].astype(jnp.int32)
    part = _sc_kernel(i1, i2, U_w, V_w, bias_U, bias_V)
    part = part.reshape(NC, BATCH)
    return (part[0] + part[1]).reshape(BATCH, 1)
